# baseline (speedup 1.0000x reference)
import jax
import jax.numpy as jnp
from jax import lax
from jax.experimental import pallas as pl
from jax.experimental.pallas import tpu as pltpu
from jax.experimental.pallas import tpu_sc as plsc

F32 = jnp.float32
I32 = jnp.int32

_N = 50000
_E = 400000
_D = 128
_H = 128

_CHUNK = 9216
_NPASS = 3
_TRASH = 128
_EB = 80
_ROWS_PER_TILE = _CHUNK // 16
_ZROWS_PER_TILE = (_CHUNK + _TRASH) // 16
_CROWS = 512

_BLK = 400
_GRID = _N // _BLK


def _dot(a, b):
    return jnp.dot(a, b, preferred_element_type=F32)


def _full(shape):
    return pl.BlockSpec(shape, lambda i: (0, 0))


def _rows(width):
    return pl.BlockSpec((_BLK, width), lambda i: (i, 0))


def _event_enc_body(x, W1, b1, W2, b2, gW1, gb1, gW2, gb2, ofW, ofb,
                    h_out, a_out, hof_out):
    h1 = jnp.maximum(_dot(x[...], W1[...]) + b1[...], 0.0)
    h2 = jnp.maximum(_dot(h1, W2[...]) + b2[...], 0.0)
    h_out[...] = h2
    g1 = jnp.maximum(_dot(h2, gW1[...]) + gb1[...], 0.0)
    a_out[...] = jax.nn.sigmoid(_dot(g1, gW2[...]) + gb2[...])
    hof_out[...] = _dot(h2, ofW[...]) + ofb[...]


def _event_enc(x, W1, b1, W2, b2, gW1, gb1, gW2, gb2, ofW, ofb):
    return pl.pallas_call(
        _event_enc_body,
        grid=(_GRID,),
        in_specs=[_rows(_D), _full((_D, _H)), _full((1, _H)), _full((_H, _H)),
                  _full((1, _H)), _full((_H, 32)), _full((1, 32)),
                  _full((32, 1)), _full((1, 1)), _full((_H, _H)),
                  _full((1, _H))],
        out_specs=[_rows(_H), _rows(1), _rows(_H)],
        out_shape=[jax.ShapeDtypeStruct((_N, _H), F32),
                   jax.ShapeDtypeStruct((_N, 1), F32),
                   jax.ShapeDtypeStruct((_N, _H), F32)],
    )(x, W1, b1, W2, b2, gW1, gb1, gW2, gb2, ofW, ofb)


def _firm_enc_body(x, W1, b1, W2, b2, rW, rb, sW, sb, h_out, hr_out, hs_out):
    h1 = jnp.maximum(_dot(x[...], W1[...]) + b1[...], 0.0)
    h2 = jnp.maximum(_dot(h1, W2[...]) + b2[...], 0.0)
    h_out[...] = h2
    hr_out[...] = _dot(h2, rW[...]) + rb[...]
    hs_out[...] = _dot(h2, sW[...]) + sb[...]


def _firm_enc(x, W1, b1, W2, b2, rW, rb, sW, sb):
    return pl.pallas_call(
        _firm_enc_body,
        grid=(_GRID,),
        in_specs=[_rows(_D), _full((_D, _H)), _full((1, _H)), _full((_H, _H)),
                  _full((1, _H)), _full((_H, _H)), _full((1, _H)),
                  _full((_H, _H)), _full((1, _H))],
        out_specs=[_rows(_H), _rows(_H), _rows(_H)],
        out_shape=[jax.ShapeDtypeStruct((_N, _H), F32),
                   jax.ShapeDtypeStruct((_N, _H), F32),
                   jax.ShapeDtypeStruct((_N, _H), F32)],
    )(x, W1, b1, W2, b2, rW, rb, sW, sb)


def _ln(m, g, b):
    mu = jnp.mean(m, axis=1, keepdims=True)
    var = jnp.mean((m - mu) ** 2, axis=1, keepdims=True)
    return (m - mu) / jnp.sqrt(var + 1e-5) * g + b


def _event_post_body(S, C, he, a, g, b, hW, hb, hm_out, out_out):
    m = S[...] / jnp.maximum(C[...], 1.0)
    hn = _ln(m, g[...], b[...])
    av = a[...]
    hm = av * he[...] + (1.0 - av) * hn
    hm_out[...] = hm
    out_out[...] = _dot(jnp.maximum(hm, 0.0), hW[...]) + hb[...]


def _event_post(S, C, he, a, g, b, hW, hb):
    return pl.pallas_call(
        _event_post_body,
        grid=(_GRID,),
        in_specs=[_rows(_H), _rows(1), _rows(_H), _rows(1),
                  _full((1, _H)), _full((1, _H)), _full((_H, 1)),
                  _full((1, 1))],
        out_specs=[_rows(_H), _rows(1)],
        out_shape=[jax.ShapeDtypeStruct((_N, _H), F32),
                   jax.ShapeDtypeStruct((_N, 1), F32)],
    )(S, C, he, a, g, b, hW, hb)


def _firm_post_body(So, Co, Ss, Cs, hf, g, b, out):
    m = (So[...] / jnp.maximum(Co[...], 1.0)
         + Ss[...] / jnp.maximum(Cs[...], 1.0))
    out[...] = hf[...] + _ln(m, g[...], b[...])


def _firm_post(So, Co, Ss, Cs, hf, g, b):
    return pl.pallas_call(
        _firm_post_body,
        grid=(_GRID,),
        in_specs=[_rows(_H), _rows(1), _rows(_H), _rows(1),
                  _rows(_H), _full((1, _H)), _full((1, _H))],
        out_specs=_rows(_H),
        out_shape=jax.ShapeDtypeStruct((_N, _H), F32),
    )(So, Co, Ss, Cs, hf, g, b)


def _seg_body(h_hbm, src_hbm, dst_hbm, w_hbm, out_hbm, cnt_hbm,
              acc, cacc, sbuf, dbuf, wbuf, dl, dlc, rows, stage, stage2, sem):
    c = lax.axis_index("c")
    sid = lax.axis_index("s")
    nb = jnp.where(sid < 8, 313, 312)
    sb0 = sid * 312 + jnp.minimum(sid, 8)
    iota = lax.iota(I32, 16)
    one = jnp.float32(1.0)
    zero = jnp.float32(0.0)
    zeros = jnp.zeros((16,), F32)

    def z2(r, carry):
        for s in range(8):
            stage2[r, pl.ds(s * 16, 16)] = zeros
        return carry
    lax.fori_loop(0, _EB, z2, 0)

    def pass_body(p, pcarry):
        lo = (2 * p + c) * _CHUNK
        do_cnt = (c == 0) & (p == 0)
        def zrow(r, carry):
            for s in range(8):
                stage[r, pl.ds(s * 16, 16)] = zeros
            return carry
        lax.fori_loop(0, _EB, zrow, 0)
        zbase = sid * _ZROWS_PER_TILE
        for k3 in range(_ZROWS_PER_TILE // _EB):
            pltpu.sync_copy(stage, acc.at[pl.ds(zbase + k3 * _EB, _EB)])
        rem = _ZROWS_PER_TILE % _EB
        if rem:
            pltpu.sync_copy(stage.at[pl.ds(0, rem)],
                            acc.at[pl.ds(zbase + (_ZROWS_PER_TILE // _EB) * _EB,
                                         rem)])
        @pl.when(do_cnt)
        def _():
            pltpu.sync_copy(stage.at[pl.ds(0, _CROWS // 16)],
                            cacc.at[pl.ds(sid * (_CROWS // 16), _CROWS // 16)])

        plsc.subcore_barrier()

        def batch_body(b, carry):
            ebase = (sb0 + b) * _EB
            pltpu.sync_copy(src_hbm.at[pl.ds(ebase, _EB)], sbuf)
            pltpu.sync_copy(dst_hbm.at[pl.ds(ebase, _EB)], dbuf)
            pltpu.sync_copy(w_hbm.at[pl.ds(ebase, _EB)], wbuf)
            pltpu.async_copy(h_hbm.at[sbuf], rows, sem).wait()
            for k in range(_EB // 16):
                dv = dbuf[pl.ds(k * 16, 16)]
                m = (dv >= lo) & (dv < lo + _CHUNK)
                trash = _CHUNK + ((k * 16 + iota) & (_TRASH - 1))
                dl[pl.ds(k * 16, 16)] = jnp.where(m, dv - lo, trash)

            def group_body(k, cy):
                wv = wbuf[pl.ds(k * 16, 16)]
                for j in range(16):
                    g = k * 16 + j
                    wspl = jnp.full((16,), wv[j], F32)
                    for s in range(8):
                        stage[g, pl.ds(s * 16, 16)] = rows[g, pl.ds(s * 16, 16)] * wspl
                return cy
            lax.fori_loop(0, _EB // 16, group_body, 0)
            pltpu.sync_copy(stage, acc.at[dl], add=True)

            @pl.when(do_cnt)
            def _():
                def conehot(k, cy):
                    dv = dbuf[pl.ds(k * 16, 16)]
                    dlc[pl.ds(k * 16, 16)] = dv >> 7
                    sv = ((dv & 127) >> 4) << 4
                    lv = dv & 15
                    for j in range(16):
                        stage2[k * 16 + j, pl.ds(sv[j], 16)] = jnp.where(
                            iota == lv[j], one, zero)
                    return cy
                lax.fori_loop(0, _EB // 16, conehot, 0)
                pltpu.sync_copy(stage2, cacc.at[dlc], add=True)
                def cclear(k, cy):
                    dv = dbuf[pl.ds(k * 16, 16)]
                    sv = ((dv & 127) >> 4) << 4
                    for j in range(16):
                        stage2[k * 16 + j, pl.ds(sv[j], 16)] = zeros
                    return cy
                lax.fori_loop(0, _EB // 16, cclear, 0)
            return carry
        lax.fori_loop(0, nb, batch_body, 0)

        plsc.subcore_barrier()
        dbase = sid * _ROWS_PER_TILE
        pltpu.sync_copy(acc.at[pl.ds(dbase, _ROWS_PER_TILE)],
                        out_hbm.at[pl.ds(lo + dbase, _ROWS_PER_TILE)])

        @pl.when(do_cnt)
        def _():
            pltpu.sync_copy(cacc.at[pl.ds(sid * (_CROWS // 16), _CROWS // 16)],
                            cnt_hbm.at[pl.ds(sid * (_CROWS // 16), _CROWS // 16)])
        plsc.subcore_barrier()
        return pcarry

    lax.fori_loop(0, _NPASS, pass_body, 0)


def _seg_agg(h, src, dst, w):
    mesh = plsc.VectorSubcoreMesh(core_axis_name="c", subcore_axis_name="s")
    k = pl.kernel(
        _seg_body,
        mesh=mesh,
        out_type=[jax.ShapeDtypeStruct((2 * _NPASS * _CHUNK, _H), F32),
                  jax.ShapeDtypeStruct((_CROWS, _H), F32)],
        scratch_types=[
            pltpu.VMEM_SHARED((_CHUNK + _TRASH, _H), F32),
            pltpu.VMEM_SHARED((_CROWS, _H), F32),
            pltpu.VMEM((_EB,), I32),
            pltpu.VMEM((_EB,), I32),
            pltpu.VMEM((_EB,), F32),
            pltpu.VMEM((_EB,), I32),
            pltpu.VMEM((_EB,), I32),
            pltpu.VMEM((_EB, _H), F32),
            pltpu.VMEM((_EB, _H), F32),
            pltpu.VMEM((_EB, _H), F32),
            pltpu.SemaphoreType.DMA,
        ],
    )
    return k(h, src, dst, w)


def kernel(event_x, firm_x, ei_rev, ei_of, ei_sim, ea_rev, ea_of, ea_sim, params):
    p = params

    def r1(v):
        return v.reshape(1, -1).astype(F32)

    he, alpha, h_of = _event_enc(event_x, p['ev_W1'], r1(p['ev_b1']),
                                 p['ev_W2'], r1(p['ev_b2']),
                                 p['g_W1'], r1(p['g_b1']),
                                 p['g_W2'], r1(p['g_b2']),
                                 p['of_W'], r1(p['of_b']))
    hf, h_rev, h_sim = _firm_enc(firm_x, p['fm_W1'], r1(p['fm_b1']),
                                 p['fm_W2'], r1(p['fm_b2']),
                                 p['rev_W'], r1(p['rev_b']),
                                 p['sim_W'], r1(p['sim_b']))

    def edges(ei, ea):
        return (ei[0].astype(I32), ei[1].astype(I32), ea[:, 0].astype(F32))

    s_rev, d_rev, w_rev = edges(ei_rev, ea_rev)
    s_of, d_of, w_of = edges(ei_of, ea_of)
    s_sim, d_sim, w_sim = edges(ei_sim, ea_sim)

    S_rev, c_rev = _seg_agg(h_rev, s_rev, d_rev, w_rev)
    S_of, c_of = _seg_agg(h_of, s_of, d_of, w_of)
    S_sim, c_sim = _seg_agg(h_sim, s_sim, d_sim, w_sim)

    def cn(cm):
        return cm.reshape(-1)[:_N].reshape(_N, 1)

    h_mix, out2d = _event_post(S_rev[:_N], cn(c_rev), he, alpha,
                               r1(p['ev_ln_g']), r1(p['ev_ln_b']),
                               p['head_W'], r1(p['head_b']))
    f_out = _firm_post(S_of[:_N], cn(c_of), S_sim[:_N], cn(c_sim), hf,
                       r1(p['fm_ln_g']), r1(p['fm_ln_b']))

    return out2d[:, 0], alpha, h_mix, f_out


# speedup vs baseline: 1.2585x; 1.2585x over previous
import jax
import jax.numpy as jnp
from jax import lax
from jax.experimental import pallas as pl
from jax.experimental.pallas import tpu as pltpu
from jax.experimental.pallas import tpu_sc as plsc

F32 = jnp.float32
I32 = jnp.int32

_N = 50000
_E = 400000
_D = 128
_H = 128

_CHUNK = 12800
_NPASS = 2
_TRASH = 128
_EB = 64
_ROWS_PER_TILE = _CHUNK // 16
_ZROWS_PER_TILE = (_CHUNK + _TRASH) // 16
_CROWS = 512

_BLK = 400
_GRID = _N // _BLK


def _dot(a, b):
    return jnp.dot(a, b, preferred_element_type=F32)


def _full(shape):
    return pl.BlockSpec(shape, lambda i: (0, 0))


def _rows(width):
    return pl.BlockSpec((_BLK, width), lambda i: (i, 0))


def _event_enc_body(x, W1, b1, W2, b2, gW1, gb1, gW2, gb2, ofW, ofb,
                    h_out, a_out, hof_out):
    h1 = jnp.maximum(_dot(x[...], W1[...]) + b1[...], 0.0)
    h2 = jnp.maximum(_dot(h1, W2[...]) + b2[...], 0.0)
    h_out[...] = h2
    g1 = jnp.maximum(_dot(h2, gW1[...]) + gb1[...], 0.0)
    a_out[...] = jax.nn.sigmoid(_dot(g1, gW2[...]) + gb2[...])
    hof_out[...] = _dot(h2, ofW[...]) + ofb[...]


def _event_enc(x, W1, b1, W2, b2, gW1, gb1, gW2, gb2, ofW, ofb):
    return pl.pallas_call(
        _event_enc_body,
        grid=(_GRID,),
        in_specs=[_rows(_D), _full((_D, _H)), _full((1, _H)), _full((_H, _H)),
                  _full((1, _H)), _full((_H, 32)), _full((1, 32)),
                  _full((32, 1)), _full((1, 1)), _full((_H, _H)),
                  _full((1, _H))],
        out_specs=[_rows(_H), _rows(1), _rows(_H)],
        out_shape=[jax.ShapeDtypeStruct((_N, _H), F32),
                   jax.ShapeDtypeStruct((_N, 1), F32),
                   jax.ShapeDtypeStruct((_N, _H), F32)],
    )(x, W1, b1, W2, b2, gW1, gb1, gW2, gb2, ofW, ofb)


def _firm_enc_body(x, W1, b1, W2, b2, rW, rb, sW, sb, h_out, hr_out, hs_out):
    h1 = jnp.maximum(_dot(x[...], W1[...]) + b1[...], 0.0)
    h2 = jnp.maximum(_dot(h1, W2[...]) + b2[...], 0.0)
    h_out[...] = h2
    hr_out[...] = _dot(h2, rW[...]) + rb[...]
    hs_out[...] = _dot(h2, sW[...]) + sb[...]


def _firm_enc(x, W1, b1, W2, b2, rW, rb, sW, sb):
    return pl.pallas_call(
        _firm_enc_body,
        grid=(_GRID,),
        in_specs=[_rows(_D), _full((_D, _H)), _full((1, _H)), _full((_H, _H)),
                  _full((1, _H)), _full((_H, _H)), _full((1, _H)),
                  _full((_H, _H)), _full((1, _H))],
        out_specs=[_rows(_H), _rows(_H), _rows(_H)],
        out_shape=[jax.ShapeDtypeStruct((_N, _H), F32),
                   jax.ShapeDtypeStruct((_N, _H), F32),
                   jax.ShapeDtypeStruct((_N, _H), F32)],
    )(x, W1, b1, W2, b2, rW, rb, sW, sb)


def _ln(m, g, b):
    mu = jnp.mean(m, axis=1, keepdims=True)
    var = jnp.mean((m - mu) ** 2, axis=1, keepdims=True)
    return (m - mu) / jnp.sqrt(var + 1e-5) * g + b


def _event_post_body(S, C, he, a, g, b, hW, hb, hm_out, out_out):
    m = S[...] / jnp.maximum(C[...], 1.0)
    hn = _ln(m, g[...], b[...])
    av = a[...]
    hm = av * he[...] + (1.0 - av) * hn
    hm_out[...] = hm
    out_out[...] = _dot(jnp.maximum(hm, 0.0), hW[...]) + hb[...]


def _event_post(S, C, he, a, g, b, hW, hb):
    return pl.pallas_call(
        _event_post_body,
        grid=(_GRID,),
        in_specs=[_rows(_H), _rows(1), _rows(_H), _rows(1),
                  _full((1, _H)), _full((1, _H)), _full((_H, 1)),
                  _full((1, 1))],
        out_specs=[_rows(_H), _rows(1)],
        out_shape=[jax.ShapeDtypeStruct((_N, _H), F32),
                   jax.ShapeDtypeStruct((_N, 1), F32)],
    )(S, C, he, a, g, b, hW, hb)


def _firm_post_body(So, Co, Ss, Cs, hf, g, b, out):
    m = (So[...] / jnp.maximum(Co[...], 1.0)
         + Ss[...] / jnp.maximum(Cs[...], 1.0))
    out[...] = hf[...] + _ln(m, g[...], b[...])


def _firm_post(So, Co, Ss, Cs, hf, g, b):
    return pl.pallas_call(
        _firm_post_body,
        grid=(_GRID,),
        in_specs=[_rows(_H), _rows(1), _rows(_H), _rows(1),
                  _rows(_H), _full((1, _H)), _full((1, _H))],
        out_specs=_rows(_H),
        out_shape=jax.ShapeDtypeStruct((_N, _H), F32),
    )(So, Co, Ss, Cs, hf, g, b)


def _seg_body(h_hbm, src_hbm, dst_hbm, w_hbm, out_hbm, cnt_hbm,
              acc, cacc, sbuf, dbuf, wbuf, dl, dlc0, dlc1, rows, stage, stage2,
              sem):
    c = lax.axis_index("c")
    sid = lax.axis_index("s")
    nb = jnp.where(sid < 10, 391, 390)
    sb0 = sid * 390 + jnp.minimum(sid, 10)
    iota = lax.iota(I32, 16)
    one = jnp.float32(1.0)
    zero = jnp.float32(0.0)
    zeros = jnp.zeros((16,), F32)

    def z2(r, carry):
        for s in range(8):
            stage2[r, pl.ds(s * 16, 16)] = zeros
        return carry
    lax.fori_loop(0, _EB // 2, z2, 0)

    def pass_body(p, pcarry):
        lo = (2 * p + c) * _CHUNK
        do_cnt = (c == 0) & (p == 0)
        def zrow(r, carry):
            for s in range(8):
                stage[r, pl.ds(s * 16, 16)] = zeros
            return carry
        lax.fori_loop(0, _EB, zrow, 0)
        zbase = sid * _ZROWS_PER_TILE
        for k3 in range(_ZROWS_PER_TILE // _EB):
            pltpu.sync_copy(stage, acc.at[pl.ds(zbase + k3 * _EB, _EB)])
        rem = _ZROWS_PER_TILE % _EB
        if rem:
            pltpu.sync_copy(stage.at[pl.ds(0, rem)],
                            acc.at[pl.ds(zbase + (_ZROWS_PER_TILE // _EB) * _EB,
                                         rem)])
        @pl.when(do_cnt)
        def _():
            pltpu.sync_copy(stage.at[pl.ds(0, _CROWS // 16)],
                            cacc.at[pl.ds(sid * (_CROWS // 16), _CROWS // 16)])

        plsc.subcore_barrier()

        def batch_body(b, carry):
            ebase = (sb0 + b) * _EB
            pltpu.sync_copy(src_hbm.at[pl.ds(ebase, _EB)], sbuf)
            pltpu.sync_copy(dst_hbm.at[pl.ds(ebase, _EB)], dbuf)
            pltpu.sync_copy(w_hbm.at[pl.ds(ebase, _EB)], wbuf)
            pltpu.async_copy(h_hbm.at[sbuf], rows, sem).wait()
            for k in range(_EB // 16):
                dv = dbuf[pl.ds(k * 16, 16)]
                m = (dv >= lo) & (dv < lo + _CHUNK)
                trash = _CHUNK + ((k * 16 + iota) & (_TRASH - 1))
                dl[pl.ds(k * 16, 16)] = jnp.where(m, dv - lo, trash)

            def group_body(k, cy):
                wv = wbuf[pl.ds(k * 16, 16)]
                for j in range(16):
                    g = k * 16 + j
                    wspl = jnp.full((16,), wv[j], F32)
                    for s in range(8):
                        stage[g, pl.ds(s * 16, 16)] = rows[g, pl.ds(s * 16, 16)] * wspl
                return cy
            lax.fori_loop(0, _EB // 16, group_body, 0)
            pltpu.sync_copy(stage, acc.at[dl], add=True)

            @pl.when(do_cnt)
            def _():
                for h, dlc in ((0, dlc0), (1, dlc1)):
                    for k in range(_EB // 32):
                        kk = h * (_EB // 32) + k
                        dv = dbuf[pl.ds(kk * 16, 16)]
                        dlc[pl.ds(k * 16, 16)] = dv >> 7
                        sv = ((dv & 127) >> 4) << 4
                        lv = dv & 15
                        for j in range(16):
                            stage2[k * 16 + j, pl.ds(sv[j], 16)] = jnp.where(
                                iota == lv[j], one, zero)
                    pltpu.sync_copy(stage2, cacc.at[dlc], add=True)
                    for k in range(_EB // 32):
                        kk = h * (_EB // 32) + k
                        dv = dbuf[pl.ds(kk * 16, 16)]
                        sv = ((dv & 127) >> 4) << 4
                        for j in range(16):
                            stage2[k * 16 + j, pl.ds(sv[j], 16)] = zeros
            return carry
        lax.fori_loop(0, nb, batch_body, 0)

        plsc.subcore_barrier()
        dbase = sid * _ROWS_PER_TILE
        pltpu.sync_copy(acc.at[pl.ds(dbase, _ROWS_PER_TILE)],
                        out_hbm.at[pl.ds(lo + dbase, _ROWS_PER_TILE)])

        @pl.when(do_cnt)
        def _():
            pltpu.sync_copy(cacc.at[pl.ds(sid * (_CROWS // 16), _CROWS // 16)],
                            cnt_hbm.at[pl.ds(sid * (_CROWS // 16), _CROWS // 16)])
        plsc.subcore_barrier()
        return pcarry

    lax.fori_loop(0, _NPASS, pass_body, 0)


def _seg_agg(h, src, dst, w):
    mesh = plsc.VectorSubcoreMesh(core_axis_name="c", subcore_axis_name="s")
    k = pl.kernel(
        _seg_body,
        mesh=mesh,
        out_type=[jax.ShapeDtypeStruct((2 * _NPASS * _CHUNK, _H), F32),
                  jax.ShapeDtypeStruct((_CROWS, _H), F32)],
        scratch_types=[
            pltpu.VMEM_SHARED((_CHUNK + _TRASH, _H), F32),
            pltpu.VMEM_SHARED((_CROWS, _H), F32),
            pltpu.VMEM((_EB,), I32),
            pltpu.VMEM((_EB,), I32),
            pltpu.VMEM((_EB,), F32),
            pltpu.VMEM((_EB,), I32),
            pltpu.VMEM((_EB // 2,), I32),
            pltpu.VMEM((_EB // 2,), I32),
            pltpu.VMEM((_EB, _H), F32),
            pltpu.VMEM((_EB, _H), F32),
            pltpu.VMEM((_EB // 2, _H), F32),
            pltpu.SemaphoreType.DMA,
        ],
    )
    return k(h, src, dst, w)


def kernel(event_x, firm_x, ei_rev, ei_of, ei_sim, ea_rev, ea_of, ea_sim, params):
    p = params

    def r1(v):
        return v.reshape(1, -1).astype(F32)

    he, alpha, h_of = _event_enc(event_x, p['ev_W1'], r1(p['ev_b1']),
                                 p['ev_W2'], r1(p['ev_b2']),
                                 p['g_W1'], r1(p['g_b1']),
                                 p['g_W2'], r1(p['g_b2']),
                                 p['of_W'], r1(p['of_b']))
    hf, h_rev, h_sim = _firm_enc(firm_x, p['fm_W1'], r1(p['fm_b1']),
                                 p['fm_W2'], r1(p['fm_b2']),
                                 p['rev_W'], r1(p['rev_b']),
                                 p['sim_W'], r1(p['sim_b']))

    def edges(ei, ea):
        return (ei[0].astype(I32), ei[1].astype(I32), ea[:, 0].astype(F32))

    s_rev, d_rev, w_rev = edges(ei_rev, ea_rev)
    s_of, d_of, w_of = edges(ei_of, ea_of)
    s_sim, d_sim, w_sim = edges(ei_sim, ea_sim)

    S_rev, c_rev = _seg_agg(h_rev, s_rev, d_rev, w_rev)
    S_of, c_of = _seg_agg(h_of, s_of, d_of, w_of)
    S_sim, c_sim = _seg_agg(h_sim, s_sim, d_sim, w_sim)

    def cn(cm):
        return cm.reshape(-1)[:_N].reshape(_N, 1)

    h_mix, out2d = _event_post(S_rev[:_N], cn(c_rev), he, alpha,
                               r1(p['ev_ln_g']), r1(p['ev_ln_b']),
                               p['head_W'], r1(p['head_b']))
    f_out = _firm_post(S_of[:_N], cn(c_of), S_sim[:_N], cn(c_sim), hf,
                       r1(p['fm_ln_g']), r1(p['fm_ln_b']))

    return out2d[:, 0], alpha, h_mix, f_out


# speedup vs baseline: 1.7982x; 1.4288x over previous
import jax
import jax.numpy as jnp
from jax import lax
from jax.experimental import pallas as pl
from jax.experimental.pallas import tpu as pltpu
from jax.experimental.pallas import tpu_sc as plsc

F32 = jnp.float32
I32 = jnp.int32

_N = 50000
_E = 400000
_D = 128
_H = 128

_CHUNK = 12800
_NPASS = 2
_TRASH = 128
_EB = 64
_ROWS_PER_TILE = _CHUNK // 16
_ZROWS_PER_TILE = (_CHUNK + _TRASH) // 16
_CROWS = 512

_BLK = 400
_GRID = _N // _BLK


def _dot(a, b):
    return jnp.dot(a, b, preferred_element_type=F32)


def _full(shape):
    return pl.BlockSpec(shape, lambda i: (0, 0))


def _rows(width):
    return pl.BlockSpec((_BLK, width), lambda i: (i, 0))


def _event_enc_body(x, W1, b1, W2, b2, gW1, gb1, gW2, gb2, ofW, ofb,
                    h_out, a_out, hof_out):
    h1 = jnp.maximum(_dot(x[...], W1[...]) + b1[...], 0.0)
    h2 = jnp.maximum(_dot(h1, W2[...]) + b2[...], 0.0)
    h_out[...] = h2
    g1 = jnp.maximum(_dot(h2, gW1[...]) + gb1[...], 0.0)
    a_out[...] = jax.nn.sigmoid(_dot(g1, gW2[...]) + gb2[...])
    hof_out[...] = _dot(h2, ofW[...]) + ofb[...]


def _event_enc(x, W1, b1, W2, b2, gW1, gb1, gW2, gb2, ofW, ofb):
    return pl.pallas_call(
        _event_enc_body,
        grid=(_GRID,),
        in_specs=[_rows(_D), _full((_D, _H)), _full((1, _H)), _full((_H, _H)),
                  _full((1, _H)), _full((_H, 32)), _full((1, 32)),
                  _full((32, 1)), _full((1, 1)), _full((_H, _H)),
                  _full((1, _H))],
        out_specs=[_rows(_H), _rows(1), _rows(_H)],
        out_shape=[jax.ShapeDtypeStruct((_N, _H), F32),
                   jax.ShapeDtypeStruct((_N, 1), F32),
                   jax.ShapeDtypeStruct((_N, _H), F32)],
    )(x, W1, b1, W2, b2, gW1, gb1, gW2, gb2, ofW, ofb)


def _firm_enc_body(x, W1, b1, W2, b2, rW, rb, sW, sb, h_out, hr_out, hs_out):
    h1 = jnp.maximum(_dot(x[...], W1[...]) + b1[...], 0.0)
    h2 = jnp.maximum(_dot(h1, W2[...]) + b2[...], 0.0)
    h_out[...] = h2
    hr_out[...] = _dot(h2, rW[...]) + rb[...]
    hs_out[...] = _dot(h2, sW[...]) + sb[...]


def _firm_enc(x, W1, b1, W2, b2, rW, rb, sW, sb):
    return pl.pallas_call(
        _firm_enc_body,
        grid=(_GRID,),
        in_specs=[_rows(_D), _full((_D, _H)), _full((1, _H)), _full((_H, _H)),
                  _full((1, _H)), _full((_H, _H)), _full((1, _H)),
                  _full((_H, _H)), _full((1, _H))],
        out_specs=[_rows(_H), _rows(_H), _rows(_H)],
        out_shape=[jax.ShapeDtypeStruct((_N, _H), F32),
                   jax.ShapeDtypeStruct((_N, _H), F32),
                   jax.ShapeDtypeStruct((_N, _H), F32)],
    )(x, W1, b1, W2, b2, rW, rb, sW, sb)


def _ln(m, g, b):
    mu = jnp.mean(m, axis=1, keepdims=True)
    var = jnp.mean((m - mu) ** 2, axis=1, keepdims=True)
    return (m - mu) / jnp.sqrt(var + 1e-5) * g + b


def _event_post_body(S, C, he, a, g, b, hW, hb, hm_out, out_out):
    m = S[...] / jnp.maximum(C[...], 1.0)
    hn = _ln(m, g[...], b[...])
    av = a[...]
    hm = av * he[...] + (1.0 - av) * hn
    hm_out[...] = hm
    out_out[...] = _dot(jnp.maximum(hm, 0.0), hW[...]) + hb[...]


def _event_post(S, C, he, a, g, b, hW, hb):
    return pl.pallas_call(
        _event_post_body,
        grid=(_GRID,),
        in_specs=[_rows(_H), _rows(1), _rows(_H), _rows(1),
                  _full((1, _H)), _full((1, _H)), _full((_H, 1)),
                  _full((1, 1))],
        out_specs=[_rows(_H), _rows(1)],
        out_shape=[jax.ShapeDtypeStruct((_N, _H), F32),
                   jax.ShapeDtypeStruct((_N, 1), F32)],
    )(S, C, he, a, g, b, hW, hb)


def _firm_post_body(So, Co, Ss, Cs, hf, g, b, out):
    m = (So[...] / jnp.maximum(Co[...], 1.0)
         + Ss[...] / jnp.maximum(Cs[...], 1.0))
    out[...] = hf[...] + _ln(m, g[...], b[...])


def _firm_post(So, Co, Ss, Cs, hf, g, b):
    return pl.pallas_call(
        _firm_post_body,
        grid=(_GRID,),
        in_specs=[_rows(_H), _rows(1), _rows(_H), _rows(1),
                  _rows(_H), _full((1, _H)), _full((1, _H))],
        out_specs=_rows(_H),
        out_shape=jax.ShapeDtypeStruct((_N, _H), F32),
    )(So, Co, Ss, Cs, hf, g, b)


def _seg_body(h_hbm, src_hbm, dst_hbm, w_hbm, out_hbm, cnt_hbm,
              acc, cacc, sbuf0, dbuf0, wbuf0, sbuf1, dbuf1, wbuf1,
              dl, dlc0, dlc1, rows0, rows1, stage2, semA, semB):
    c = lax.axis_index("c")
    sid = lax.axis_index("s")
    nb = jnp.where(sid < 10, 391, 390)
    sb0 = sid * 390 + jnp.minimum(sid, 10)
    iota = lax.iota(I32, 16)
    one = jnp.float32(1.0)
    zero = jnp.float32(0.0)
    zeros = jnp.zeros((16,), F32)

    def z2(r, carry):
        for s in range(8):
            stage2[r, pl.ds(s * 16, 16)] = zeros
        return carry
    lax.fori_loop(0, _EB // 2, z2, 0)

    def fetch(j, sbuf, dbuf, wbuf, rows, sem):
        ebase = j * _EB
        pltpu.sync_copy(src_hbm.at[pl.ds(ebase, _EB)], sbuf)
        pltpu.sync_copy(dst_hbm.at[pl.ds(ebase, _EB)], dbuf)
        pltpu.sync_copy(w_hbm.at[pl.ds(ebase, _EB)], wbuf)
        pltpu.async_copy(h_hbm.at[sbuf], rows, sem)

    def drain(sbuf, rows, sem):
        pltpu.make_async_copy(h_hbm.at[sbuf], rows, sem).wait()

    def process(dbuf, wbuf, rows, lo, do_cnt):
        for k in range(_EB // 16):
            dv = dbuf[pl.ds(k * 16, 16)]
            m = (dv >= lo) & (dv < lo + _CHUNK)
            trash = _CHUNK + ((k * 16 + iota) & (_TRASH - 1))
            dl[pl.ds(k * 16, 16)] = jnp.where(m, dv - lo, trash)

        def group_body(k, cy):
            wv = wbuf[pl.ds(k * 16, 16)]
            for j in range(16):
                g = k * 16 + j
                wspl = jnp.full((16,), wv[j], F32)
                for s in range(8):
                    rows[g, pl.ds(s * 16, 16)] = rows[g, pl.ds(s * 16, 16)] * wspl
            return cy
        lax.fori_loop(0, _EB // 16, group_body, 0)
        pltpu.sync_copy(rows, acc.at[dl], add=True)

        @pl.when(do_cnt)
        def _():
            for h, dlc in ((0, dlc0), (1, dlc1)):
                for k in range(_EB // 32):
                    kk = h * (_EB // 32) + k
                    dv = dbuf[pl.ds(kk * 16, 16)]
                    dlc[pl.ds(k * 16, 16)] = dv >> 7
                    sv = ((dv & 127) >> 4) << 4
                    lv = dv & 15
                    for j in range(16):
                        stage2[k * 16 + j, pl.ds(sv[j], 16)] = jnp.where(
                            iota == lv[j], one, zero)
                pltpu.sync_copy(stage2, cacc.at[dlc], add=True)
                for k in range(_EB // 32):
                    kk = h * (_EB // 32) + k
                    dv = dbuf[pl.ds(kk * 16, 16)]
                    sv = ((dv & 127) >> 4) << 4
                    for j in range(16):
                        stage2[k * 16 + j, pl.ds(sv[j], 16)] = zeros

    def pass_body(p, pcarry):
        lo = (2 * p + c) * _CHUNK
        do_cnt = (c == 0) & (p == 0)
        zbase = sid * _ZROWS_PER_TILE
        zb = _EB // 2
        for k3 in range(_ZROWS_PER_TILE // zb):
            pltpu.sync_copy(stage2, acc.at[pl.ds(zbase + k3 * zb, zb)])
        rem = _ZROWS_PER_TILE % zb
        if rem:
            pltpu.sync_copy(stage2.at[pl.ds(0, rem)],
                            acc.at[pl.ds(zbase + (_ZROWS_PER_TILE // zb) * zb,
                                         rem)])
        @pl.when(do_cnt)
        def _():
            pltpu.sync_copy(stage2,
                            cacc.at[pl.ds(sid * (_CROWS // 16), _CROWS // 16)])

        plsc.subcore_barrier()

        fetch(sb0, sbuf0, dbuf0, wbuf0, rows0, semA)

        def pair_body(i, cy):
            b0 = sb0 + 2 * i
            drain(sbuf0, rows0, semA)
            fetch(b0 + 1, sbuf1, dbuf1, wbuf1, rows1, semB)
            process(dbuf0, wbuf0, rows0, lo, do_cnt)

            @pl.when(2 * i + 2 < nb)
            def _():
                fetch(b0 + 2, sbuf0, dbuf0, wbuf0, rows0, semA)
            drain(sbuf1, rows1, semB)
            process(dbuf1, wbuf1, rows1, lo, do_cnt)
            return cy
        lax.fori_loop(0, nb // 2, pair_body, 0)

        @pl.when((nb & 1) == 1)
        def _():
            drain(sbuf0, rows0, semA)
            process(dbuf0, wbuf0, rows0, lo, do_cnt)

        plsc.subcore_barrier()
        dbase = sid * _ROWS_PER_TILE
        pltpu.sync_copy(acc.at[pl.ds(dbase, _ROWS_PER_TILE)],
                        out_hbm.at[pl.ds(lo + dbase, _ROWS_PER_TILE)])

        @pl.when(do_cnt)
        def _():
            pltpu.sync_copy(cacc.at[pl.ds(sid * (_CROWS // 16), _CROWS // 16)],
                            cnt_hbm.at[pl.ds(sid * (_CROWS // 16), _CROWS // 16)])
        plsc.subcore_barrier()
        return pcarry

    lax.fori_loop(0, _NPASS, pass_body, 0)


def _seg_agg(h, src, dst, w):
    mesh = plsc.VectorSubcoreMesh(core_axis_name="c", subcore_axis_name="s")
    k = pl.kernel(
        _seg_body,
        mesh=mesh,
        out_type=[jax.ShapeDtypeStruct((2 * _NPASS * _CHUNK, _H), F32),
                  jax.ShapeDtypeStruct((_CROWS, _H), F32)],
        scratch_types=[
            pltpu.VMEM_SHARED((_CHUNK + _TRASH, _H), F32),
            pltpu.VMEM_SHARED((_CROWS, _H), F32),
            pltpu.VMEM((_EB,), I32),
            pltpu.VMEM((_EB,), I32),
            pltpu.VMEM((_EB,), F32),
            pltpu.VMEM((_EB,), I32),
            pltpu.VMEM((_EB,), I32),
            pltpu.VMEM((_EB,), F32),
            pltpu.VMEM((_EB,), I32),
            pltpu.VMEM((_EB // 2,), I32),
            pltpu.VMEM((_EB // 2,), I32),
            pltpu.VMEM((_EB, _H), F32),
            pltpu.VMEM((_EB, _H), F32),
            pltpu.VMEM((_EB // 2, _H), F32),
            pltpu.SemaphoreType.DMA,
            pltpu.SemaphoreType.DMA,
        ],
    )
    return k(h, src, dst, w)


def kernel(event_x, firm_x, ei_rev, ei_of, ei_sim, ea_rev, ea_of, ea_sim, params):
    p = params

    def r1(v):
        return v.reshape(1, -1).astype(F32)

    he, alpha, h_of = _event_enc(event_x, p['ev_W1'], r1(p['ev_b1']),
                                 p['ev_W2'], r1(p['ev_b2']),
                                 p['g_W1'], r1(p['g_b1']),
                                 p['g_W2'], r1(p['g_b2']),
                                 p['of_W'], r1(p['of_b']))
    hf, h_rev, h_sim = _firm_enc(firm_x, p['fm_W1'], r1(p['fm_b1']),
                                 p['fm_W2'], r1(p['fm_b2']),
                                 p['rev_W'], r1(p['rev_b']),
                                 p['sim_W'], r1(p['sim_b']))

    def edges(ei, ea):
        return (ei[0].astype(I32), ei[1].astype(I32), ea[:, 0].astype(F32))

    s_rev, d_rev, w_rev = edges(ei_rev, ea_rev)
    s_of, d_of, w_of = edges(ei_of, ea_of)
    s_sim, d_sim, w_sim = edges(ei_sim, ea_sim)

    S_rev, c_rev = _seg_agg(h_rev, s_rev, d_rev, w_rev)
    S_of, c_of = _seg_agg(h_of, s_of, d_of, w_of)
    S_sim, c_sim = _seg_agg(h_sim, s_sim, d_sim, w_sim)

    def cn(cm):
        return cm.reshape(-1)[:_N].reshape(_N, 1)

    h_mix, out2d = _event_post(S_rev[:_N], cn(c_rev), he, alpha,
                               r1(p['ev_ln_g']), r1(p['ev_ln_b']),
                               p['head_W'], r1(p['head_b']))
    f_out = _firm_post(S_of[:_N], cn(c_of), S_sim[:_N], cn(c_sim), hf,
                       r1(p['fm_ln_g']), r1(p['fm_ln_b']))

    return out2d[:, 0], alpha, h_mix, f_out
